# baseline (speedup 1.0000x reference)
import functools

import jax
import jax.numpy as jnp
import numpy as np
from jax import lax
from jax.experimental import pallas as pl
from jax.experimental.pallas import tpu as pltpu
from jax.experimental.pallas import tpu_sc as plsc

_ELECTRODE_COORDS = np.array(
    [[2, 1], [4, 1], [1, 2], [2, 2], [3, 2], [4, 2], [5, 2], [1, 3], [2, 3],
     [3, 3], [4, 3], [5, 3], [1, 4], [2, 4], [3, 4], [4, 4], [5, 4], [2, 5],
     [4, 5]], dtype=np.int32)

_N = 19
_HALF = 1024

_IDX_ROWS = np.zeros((_N, 16), dtype=np.int32)
_IDX_ROWS[:, 0] = _ELECTRODE_COORDS[:, 0]
_IDX_ROWS[:, 1] = _ELECTRODE_COORDS[:, 1]

_SC_INFO = plsc.get_sparse_core_info()
_NC = _SC_INFO.num_cores
_NS = _SC_INFO.num_subcores


@functools.partial(
    pl.kernel,
    mesh=plsc.VectorSubcoreMesh(core_axis_name="c", subcore_axis_name="s"),
    out_type=jax.ShapeDtypeStruct((_N, 2, _HALF), jnp.float32),
    scratch_types=[
        pltpu.VMEM((16,), jnp.int32),
        pltpu.VMEM((2, _HALF), jnp.float32),
        pltpu.SemaphoreType.DMA,
    ],
)
def _pe_gather(table_hbm, idx_hbm, out_hbm, idx_v, rows_v, sem):
    wid = lax.axis_index("s") * _NC + lax.axis_index("c")

    @pl.when(wid < _N)
    def _():
        pltpu.sync_copy(idx_hbm.at[wid], idx_v)
        pltpu.async_copy(table_hbm.at[idx_v.at[pl.ds(0, 2)]], rows_v, sem).wait()
        pltpu.sync_copy(rows_v, out_hbm.at[wid])


def kernel(x, pe):
    del x
    table = pe.reshape(pe.shape[1], pe.shape[2])
    idx = jnp.asarray(_IDX_ROWS)
    out = _pe_gather(table, idx)
    return out.reshape(1, _N, 2 * _HALF)
